# Initial kernel scaffold; baseline (speedup 1.0000x reference)
import jax
import jax.numpy as jnp
from jax.experimental import pallas as pl


def kernel(x, table):
    raise NotImplementedError("write your pallas kernel here")


# speedup vs baseline: 1.2895x; 1.2895x over previous
import functools

import jax
import jax.numpy as jnp
from jax import lax
from jax.experimental import pallas as pl
from jax.experimental.pallas import tpu as pltpu
from jax.experimental.pallas import tpu_sc as plsc

BATCH = 16384
N_FIELDS = 26
NW = 32
BPW = BATCH // NW
IDX_PW = BPW * N_FIELDS
CW = 128
NCHUNK = IDX_PW // CW
NGROUP = BPW // 16


def _emb_body(x_hbm, table_hbm, out_hbm, idx_v, vals_v, o_v, sem):
    wid = lax.axis_index("s") * 2 + lax.axis_index("c")
    pltpu.sync_copy(x_hbm.at[pl.ds(wid * NCHUNK, NCHUNK)], idx_v)
    def fire(j, carry):
        dst = vals_v.at[pl.ds(pl.multiple_of(j * CW, CW), CW)]
        pltpu.make_async_copy(table_hbm.at[idx_v.at[j]], dst, sem).start()
        return carry

    def drain(j, carry):
        dst = vals_v.at[pl.ds(pl.multiple_of(j * CW, CW), CW)]
        pltpu.make_async_copy(table_hbm.at[idx_v.at[j]], dst, sem).wait()
        return carry

    lax.fori_loop(0, NCHUNK, fire, 0)
    lax.fori_loop(0, NCHUNK, drain, 0)

    lanes = lax.iota(jnp.int32, 16)

    def group(g, carry):
        q0 = lanes * N_FIELDS + g * (16 * N_FIELDS)

        def field(k, acc):
            return acc + plsc.load_gather(vals_v, [q0 + k])

        s = lax.fori_loop(0, N_FIELDS, field, jnp.zeros((16,), jnp.float32))
        o_v[pl.ds(pl.multiple_of(g * 16, 16), 16)] = 1.0 / (1.0 + jnp.exp(-s))
        return carry

    lax.fori_loop(0, NGROUP, group, 0)
    pltpu.sync_copy(o_v, out_hbm.at[pl.ds(wid * BPW, BPW)])


def _emb_call(xf, tf):
    mesh = plsc.VectorSubcoreMesh(core_axis_name="c", subcore_axis_name="s")
    return pl.kernel(
        _emb_body,
        out_type=jax.ShapeDtypeStruct((BATCH,), jnp.float32),
        mesh=mesh,
        scratch_types=[
            pltpu.VMEM((NCHUNK, CW), jnp.int32),
            pltpu.VMEM((IDX_PW,), jnp.float32),
            pltpu.VMEM((BPW,), jnp.float32),
            pltpu.SemaphoreType.DMA,
        ],
        compiler_params=pltpu.CompilerParams(needs_layout_passes=False),
    )(xf, tf)


def kernel(x, table):
    xf = x.astype(jnp.int32).reshape(BATCH * N_FIELDS // CW, CW)
    tf = table.reshape(-1)
    return _emb_call(xf, tf).reshape(BATCH, 1)


# speedup vs baseline: 1.2918x; 1.0018x over previous
import functools

import jax
import jax.numpy as jnp
from jax import lax
from jax.experimental import pallas as pl
from jax.experimental.pallas import tpu as pltpu
from jax.experimental.pallas import tpu_sc as plsc

BATCH = 16384
N_FIELDS = 26
NW = 32
BPW = BATCH // NW
IDX_PW = BPW * N_FIELDS
CW = 128
NCHUNK = IDX_PW // CW
NGROUP = BPW // 16


def _emb_body(x_hbm, table_hbm, out_hbm, idx_v, vals_v, o_v, sem):
    wid = lax.axis_index("s") * 2 + lax.axis_index("c")
    pltpu.sync_copy(x_hbm.at[pl.ds(wid * IDX_PW, IDX_PW)], idx_v)
    pltpu.make_async_copy(table_hbm.at[idx_v], vals_v, sem).start()
    pltpu.make_async_copy(table_hbm.at[idx_v], vals_v, sem).wait()

    lanes = lax.iota(jnp.int32, 16)

    def group(g, carry):
        q0 = lanes * N_FIELDS + g * (16 * N_FIELDS)

        def field(k, acc):
            return acc + plsc.load_gather(vals_v, [q0 + k])

        s = lax.fori_loop(0, N_FIELDS, field, jnp.zeros((16,), jnp.float32))
        o_v[pl.ds(pl.multiple_of(g * 16, 16), 16)] = 1.0 / (1.0 + jnp.exp(-s))
        return carry

    lax.fori_loop(0, NGROUP, group, 0)
    pltpu.sync_copy(o_v, out_hbm.at[pl.ds(wid * BPW, BPW)])


def _emb_call(xf, tf):
    mesh = plsc.VectorSubcoreMesh(core_axis_name="c", subcore_axis_name="s")
    return pl.kernel(
        _emb_body,
        out_type=jax.ShapeDtypeStruct((BATCH,), jnp.float32),
        mesh=mesh,
        scratch_types=[
            pltpu.VMEM((IDX_PW,), jnp.int32),
            pltpu.VMEM((IDX_PW,), jnp.float32),
            pltpu.VMEM((BPW,), jnp.float32),
            pltpu.SemaphoreType.DMA,
        ],
        compiler_params=pltpu.CompilerParams(needs_layout_passes=False),
    )(xf, tf)


def kernel(x, table):
    xf = x.astype(jnp.int32).reshape(BATCH * N_FIELDS)
    tf = table.reshape(-1)
    return _emb_call(xf, tf).reshape(BATCH, 1)


# speedup vs baseline: 1.3268x; 1.0271x over previous
import functools

import jax
import jax.numpy as jnp
from jax import lax
from jax.experimental import pallas as pl
from jax.experimental.pallas import tpu as pltpu
from jax.experimental.pallas import tpu_sc as plsc

BATCH = 16384
N_FIELDS = 26
NW = 32
BPW = BATCH // NW
IDX_PW = BPW * N_FIELDS
CW = 128
NCHUNK = IDX_PW // CW
NGROUP = BPW // 16


def _emb_body(x_hbm, table_hbm, out_hbm, idx_v, vals_v, o_v, sem):
    wid = lax.axis_index("s") * 2 + lax.axis_index("c")
    pltpu.sync_copy(x_hbm.at[pl.ds(wid * IDX_PW, IDX_PW)], idx_v)
    pltpu.make_async_copy(table_hbm.at[idx_v], vals_v, sem).start()
    pltpu.make_async_copy(table_hbm.at[idx_v], vals_v, sem).wait()

    lanes = lax.iota(jnp.int32, 16)

    def group(g, carry):
        q0 = lanes * N_FIELDS + g * (16 * N_FIELDS)
        acc0 = plsc.load_gather(vals_v, [q0])
        acc1 = plsc.load_gather(vals_v, [q0 + 1])
        for k in range(2, N_FIELDS, 2):
            acc0 = acc0 + plsc.load_gather(vals_v, [q0 + k])
            acc1 = acc1 + plsc.load_gather(vals_v, [q0 + (k + 1)])
        s = acc0 + acc1
        o_v[pl.ds(pl.multiple_of(g * 16, 16), 16)] = 1.0 / (1.0 + jnp.exp(-s))
        return carry

    lax.fori_loop(0, NGROUP, group, 0)
    pltpu.sync_copy(o_v, out_hbm.at[pl.ds(wid * BPW, BPW)])


def _emb_call(xf, tf):
    mesh = plsc.VectorSubcoreMesh(core_axis_name="c", subcore_axis_name="s")
    return pl.kernel(
        _emb_body,
        out_type=jax.ShapeDtypeStruct((BATCH,), jnp.float32),
        mesh=mesh,
        scratch_types=[
            pltpu.VMEM((IDX_PW,), jnp.int32),
            pltpu.VMEM((IDX_PW,), jnp.float32),
            pltpu.VMEM((BPW,), jnp.float32),
            pltpu.SemaphoreType.DMA,
        ],
        compiler_params=pltpu.CompilerParams(needs_layout_passes=False),
    )(xf, tf)


def kernel(x, table):
    xf = x.astype(jnp.int32).reshape(BATCH * N_FIELDS)
    tf = table.reshape(-1)
    return _emb_call(xf, tf).reshape(BATCH, 1)


# speedup vs baseline: 1.3280x; 1.0009x over previous
import functools

import jax
import jax.numpy as jnp
from jax import lax
from jax.experimental import pallas as pl
from jax.experimental.pallas import tpu as pltpu
from jax.experimental.pallas import tpu_sc as plsc

BATCH = 16384
N_FIELDS = 26
NW = 32
BPW = BATCH // NW
IDX_PW = BPW * N_FIELDS
CW = 128
NCHUNK = IDX_PW // CW
NGROUP = BPW // 16


def _emb_body(x_hbm, table_hbm, out_hbm, idx_v, vals_v, o_v, sem):
    sid = lax.axis_index("s")
    wid = sid * 2 + lax.axis_index("c")
    with jax.named_scope("idx_stage"):
        pltpu.sync_copy(x_hbm.at[pl.ds(wid * IDX_PW, IDX_PW)], idx_v)
    with jax.named_scope("gather"):
        pltpu.make_async_copy(table_hbm.at[idx_v], vals_v, sem).start()
        pltpu.make_async_copy(table_hbm.at[idx_v], vals_v, sem).wait()

    lanes = lax.iota(jnp.int32, 16)

    def group(g, carry):
        q0 = lanes * N_FIELDS + g * (16 * N_FIELDS)
        acc0 = plsc.load_gather(vals_v, [q0])
        acc1 = plsc.load_gather(vals_v, [q0 + 1])
        for k in range(2, N_FIELDS, 2):
            acc0 = acc0 + plsc.load_gather(vals_v, [q0 + k])
            acc1 = acc1 + plsc.load_gather(vals_v, [q0 + (k + 1)])
        s = acc0 + acc1
        o_v[pl.ds(pl.multiple_of(g * 16, 16), 16)] = 1.0 / (1.0 + jnp.exp(-s))
        return carry

    with jax.named_scope("reduce"):
        lax.fori_loop(0, NGROUP, group, 0)
    with jax.named_scope("writeout"):
        pltpu.sync_copy(o_v, out_hbm.at[pl.ds(wid * BPW, BPW)])


def _emb_call(xf, tf):
    mesh = plsc.VectorSubcoreMesh(core_axis_name="c", subcore_axis_name="s")
    return pl.kernel(
        _emb_body,
        out_type=jax.ShapeDtypeStruct((BATCH,), jnp.float32),
        mesh=mesh,
        scratch_types=[
            pltpu.VMEM((IDX_PW,), jnp.int32),
            pltpu.VMEM((IDX_PW,), jnp.float32),
            pltpu.VMEM((BPW,), jnp.float32),
            pltpu.SemaphoreType.DMA,
        ],
        compiler_params=pltpu.CompilerParams(needs_layout_passes=False),
    )(xf, tf)


def kernel(x, table):
    xf = x.astype(jnp.int32).reshape(BATCH * N_FIELDS)
    tf = table.reshape(-1)
    return _emb_call(xf, tf).reshape(BATCH, 1)


# speedup vs baseline: 1.6240x; 1.2229x over previous
import functools

import jax
import jax.numpy as jnp
from jax import lax
from jax.experimental import pallas as pl
from jax.experimental.pallas import tpu as pltpu
from jax.experimental.pallas import tpu_sc as plsc

BATCH = 16384
N_FIELDS = 26
VOCAB = 1000000
NW = 32
BPW = BATCH // NW
IDX_PW = BPW * N_FIELDS
NGROUP = BPW // 16


def _emb_body(xt_hbm, tt_hbm, out_hbm, idx1_v, vals_v, o_v, sem):
    wid = lax.axis_index("s") * 2 + lax.axis_index("c")
    base = wid * BPW
    for f in range(N_FIELDS):
        dst = idx1_v.at[pl.ds(pl.multiple_of(f * BPW, BPW), BPW)]
        pltpu.make_async_copy(xt_hbm.at[f].at[pl.ds(base, BPW)], dst, sem).start()
    for f in range(N_FIELDS):
        dst = idx1_v.at[pl.ds(pl.multiple_of(f * BPW, BPW), BPW)]
        pltpu.make_async_copy(xt_hbm.at[f].at[pl.ds(base, BPW)], dst, sem).wait()

    tflat = tt_hbm.at[0]
    pltpu.make_async_copy(tflat.at[idx1_v], vals_v, sem).start()
    pltpu.make_async_copy(tflat.at[idx1_v], vals_v, sem).wait()

    lanes = lax.iota(jnp.int32, 16)

    def group(g, carry):
        o16 = pl.multiple_of(g * 16, 16)
        acc0 = vals_v[pl.ds(o16, 16)]
        acc1 = vals_v[pl.ds(o16 + BPW, 16)]
        for f in range(2, N_FIELDS, 2):
            acc0 = acc0 + vals_v[pl.ds(o16 + f * BPW, 16)]
            acc1 = acc1 + vals_v[pl.ds(o16 + (f + 1) * BPW, 16)]
        s = acc0 + acc1
        o_v[pl.ds(o16, 16)] = 1.0 / (1.0 + jnp.exp(-s))
        return carry

    lax.fori_loop(0, NGROUP, group, 0)
    pltpu.sync_copy(o_v, out_hbm.at[pl.ds(base, BPW)])


def _emb_call(xt, tt):
    mesh = plsc.VectorSubcoreMesh(core_axis_name="c", subcore_axis_name="s")
    return pl.kernel(
        _emb_body,
        out_type=jax.ShapeDtypeStruct((BATCH,), jnp.float32),
        mesh=mesh,
        scratch_types=[
            pltpu.VMEM((IDX_PW,), jnp.int32),
            pltpu.VMEM((IDX_PW,), jnp.float32),
            pltpu.VMEM((BPW,), jnp.float32),
            pltpu.SemaphoreType.DMA,
        ],
        compiler_params=pltpu.CompilerParams(
            needs_layout_passes=False, use_tc_tiling_on_sc=False),
    )(xt, tt)


def kernel(x, table):
    xt = x.astype(jnp.int32).T
    tt = table.T
    return _emb_call(xt, tt).reshape(BATCH, 1)


# speedup vs baseline: 2.6611x; 1.6386x over previous
import functools

import jax
import jax.numpy as jnp
from jax import lax
from jax.experimental import pallas as pl
from jax.experimental.pallas import tpu as pltpu
from jax.experimental.pallas import tpu_sc as plsc

BATCH = 16384
N_FIELDS = 26
VOCAB = 1000000
VOCABP = 1000448
NW = 32
BPW = BATCH // NW
IDX_PW = BPW * N_FIELDS
NGROUP = BPW // 16


def _emb_body(xt_hbm, tt_hbm, out_hbm, idx1_v, vals_v, o_v, sem):
    wid = lax.axis_index("s") * 2 + lax.axis_index("c")
    base = wid * BPW
    for f in range(N_FIELDS):
        dst = idx1_v.at[pl.ds(pl.multiple_of(f * BPW, BPW), BPW)]
        pltpu.make_async_copy(xt_hbm.at[f].at[pl.ds(base, BPW)], dst, sem).start()
    for f in range(N_FIELDS):
        dst = idx1_v.at[pl.ds(pl.multiple_of(f * BPW, BPW), BPW)]
        pltpu.make_async_copy(xt_hbm.at[f].at[pl.ds(base, BPW)], dst, sem).wait()

    tflat = tt_hbm.at[0]
    pltpu.make_async_copy(tflat.at[idx1_v], vals_v, sem).start()
    pltpu.make_async_copy(tflat.at[idx1_v], vals_v, sem).wait()

    lanes = lax.iota(jnp.int32, 16)

    def group(g, carry):
        o16 = pl.multiple_of(g * 16, 16)
        acc0 = vals_v[pl.ds(o16, 16)]
        acc1 = vals_v[pl.ds(o16 + BPW, 16)]
        for f in range(2, N_FIELDS, 2):
            acc0 = acc0 + vals_v[pl.ds(o16 + f * BPW, 16)]
            acc1 = acc1 + vals_v[pl.ds(o16 + (f + 1) * BPW, 16)]
        s = acc0 + acc1
        o_v[pl.ds(o16, 16)] = 1.0 / (1.0 + jnp.exp(-s))
        return carry

    lax.fori_loop(0, NGROUP, group, 0)
    pltpu.sync_copy(o_v, out_hbm.at[pl.ds(base, BPW)])


def _emb_call(xt, tt):
    mesh = plsc.VectorSubcoreMesh(core_axis_name="c", subcore_axis_name="s")
    return pl.kernel(
        _emb_body,
        out_type=jax.ShapeDtypeStruct((BATCH,), jnp.float32),
        mesh=mesh,
        scratch_types=[
            pltpu.VMEM((IDX_PW,), jnp.int32),
            pltpu.VMEM((IDX_PW,), jnp.float32),
            pltpu.VMEM((BPW,), jnp.float32),
            pltpu.SemaphoreType.DMA,
        ],
        compiler_params=pltpu.CompilerParams(
            needs_layout_passes=False, use_tc_tiling_on_sc=False),
    )(xt, tt)


def kernel(x, table):
    xt = x.astype(jnp.int32).T
    tp = jnp.pad(table, ((0, VOCABP - VOCAB), (0, 0)))
    return _emb_call(xt, tp.T).reshape(BATCH, 1)
